# Initial kernel scaffold; baseline (speedup 1.0000x reference)
import jax
import jax.numpy as jnp
from jax.experimental import pallas as pl


def kernel(x, edge_all_list, edge_idx_list, edge_droped_idx_list, params):
    raise NotImplementedError("write your pallas kernel here")


# speedup vs baseline: 5.9986x; 5.9986x over previous
import functools

import jax
import jax.numpy as jnp
from jax import lax
from jax.experimental import pallas as pl
from jax.experimental.pallas import tpu as pltpu
from jax.experimental.pallas import tpu_sc as plsc

N = 10000
T = 3
E = 320000
XD = 128
HD = 128
ZD = 64

NP = 10240
RB = 256
GRID = NP // RB
NC = 2
NS = 16
BPW = 79
NB = NC * NS * BPW
EP = NB * 128
RPT = NP // NS
RPW = NP // (NC * NS)

_mesh = plsc.VectorSubcoreMesh(core_axis_name="c", subcore_axis_name="s")


def _deg_dinv_body(d0, d1, d2, out, dacc, zvec, ones, didx, dtile, bbuf):
    c = lax.axis_index("c")
    s = lax.axis_index("s")
    zero16 = jnp.zeros((16,), jnp.float32)
    one16 = jnp.ones((16,), jnp.float32)
    for i in range(RPT // 16):
        zvec[pl.ds(i * 16, 16)] = zero16
    for i in range(8):
        ones[pl.ds(i * 16, 16)] = one16
    for t, dref in enumerate((d0, d1, d2)):
        pltpu.sync_copy(zvec, dacc.at[pl.ds(s * RPT, RPT)])
        plsc.subcore_barrier()

        def hist(j, carry):
            base = (s * (NB // NS) + j) * 128
            pltpu.sync_copy(dref.at[pl.ds(base, 128)], didx)
            pltpu.sync_copy(ones, dacc.at[didx], add=True)
            return carry

        lax.fori_loop(0, NB // NS, hist, 0)
        plsc.subcore_barrier()

        row0 = (s * NC + c) * RPW
        pltpu.sync_copy(dacc.at[pl.ds(row0, RPW)], dtile)

        def bc(k, carry):
            y = dtile[pl.ds(k * 16, 16)]
            for l in range(16):
                row = jnp.full((16,), y[l], jnp.float32)
                for c8 in range(8):
                    bbuf[pl.ds((k * 16 + l) * 128 + c8 * 16, 16)] = row
            return carry

        lax.fori_loop(0, RPW // 16, bc, 0)
        pltpu.sync_copy(bbuf,
                        out.at[pl.ds((t * NP + row0) * 128, RPW * 128)])
        plsc.subcore_barrier()


_deg_dinv = functools.partial(
    pl.kernel,
    out_type=jax.ShapeDtypeStruct((T * NP * 128,), jnp.float32),
    mesh=_mesh,
    scratch_types=[
        pltpu.VMEM_SHARED((NP,), jnp.float32),
        pltpu.VMEM((RPT,), jnp.float32),
        pltpu.VMEM((128,), jnp.float32),
        pltpu.VMEM((128,), jnp.int32),
        pltpu.VMEM((RPW,), jnp.float32),
        pltpu.VMEM((RPW * 128,), jnp.float32),
    ],
)(_deg_dinv_body)


def _make_agg(ntab):
    def body(*refs):
        tabs = refs[:ntab]
        src = refs[ntab]
        dst = refs[ntab + 1]
        outs = refs[ntab + 2:2 * ntab + 2]
        acc, sidx, didx, rows, zbuf, sem = refs[2 * ntab + 2:]
        c = lax.axis_index("c")
        s = lax.axis_index("s")
        zero16 = jnp.zeros((16,), jnp.float32)
        for i in range(16):
            for j in range(8):
                zbuf[i, pl.ds(j * 16, 16)] = zero16
        base_batch = (c * NS + s) * BPW
        for k in range(ntab):
            def zb(i, carry):
                pltpu.sync_copy(zbuf, acc.at[pl.ds(s * RPT + i * 16, 16)])
                return carry

            lax.fori_loop(0, RPT // 16, zb, 0)
            plsc.subcore_barrier()

            def eb(j, carry):
                base = (base_batch + j) * 128
                pltpu.sync_copy(src.at[pl.ds(base, 128)], sidx)
                pltpu.async_copy(tabs[k].at[sidx], rows, sem).wait()
                pltpu.sync_copy(dst.at[pl.ds(base, 128)], didx)
                pltpu.sync_copy(rows, acc.at[didx], add=True)
                return carry

            lax.fori_loop(0, BPW, eb, 0)
            plsc.subcore_barrier()
            pltpu.sync_copy(acc.at[pl.ds(s * RPT, RPT)],
                            outs[k].at[c, pl.ds(s * RPT, RPT)])
            plsc.subcore_barrier()

    return pl.kernel(
        body,
        out_type=[jax.ShapeDtypeStruct((NC, NP, 128), jnp.float32)] * ntab,
        mesh=_mesh,
        scratch_types=[
            pltpu.VMEM_SHARED((NP, 128), jnp.float32),
            pltpu.VMEM((128,), jnp.int32),
            pltpu.VMEM((128,), jnp.int32),
            pltpu.VMEM((128, 128), jnp.float32),
            pltpu.VMEM((16, 128), jnp.float32),
            pltpu.SemaphoreType.DMA,
        ],
    )


_agg1 = _make_agg(1)
_agg4 = _make_agg(4)


def _row_spec(cols):
    return pl.BlockSpec((RB, cols), lambda i: (i, 0))


def _raw_spec(cols):
    return pl.BlockSpec((NC, RB, cols), lambda i: (0, i, 0))


def _full_spec(shape):
    nd = len(shape)
    return pl.BlockSpec(shape, lambda i: (0,) * nd)


def _pre_body(x_ref, wpx, bpx, wet, wgt, o_a1, o_gxp):
    phi = jnp.maximum(
        jnp.dot(x_ref[...], wpx[...], preferred_element_type=jnp.float32)
        + bpx[...], 0.0)
    o_a1[...] = jnp.dot(phi, wet[...], preferred_element_type=jnp.float32)
    o_gxp[...] = jnp.dot(phi, wgt[...], preferred_element_type=jnp.float32)


def _s1_body(a1, h, wencb, dv, o):
    dvv = lax.rsqrt(dv[...] + 1.0)
    o[...] = dvv * (
        a1[...] + jnp.dot(h[...], wencb[...], preferred_element_type=jnp.float32))


def _s2_body(renc, xenc, dv, benc, wems, o):
    dvv = lax.rsqrt(dv[...] + 1.0)
    enc = jnp.maximum(
        dvv * (renc[0] + renc[1] + xenc[...]) + benc[...], 0.0)
    o[...] = dvv * jnp.dot(enc, wems[...], preferred_element_type=jnp.float32)


def _softplus(x):
    return jnp.maximum(x, 0.0) + jnp.log1p(jnp.exp(-jnp.abs(x)))


def _s3_body(rms, xms, dv, bems, eps, wpz, bpz, wdec, gxp, wxb, h, whz, whr,
             o_dec, o_zg, o_rg, o_xh):
    dvv = lax.rsqrt(dv[...] + 1.0)
    agg = dvv * (rms[0] + rms[1] + xms[...]) + bems[...]
    mean = agg[:, :ZD]
    std = _softplus(agg[:, ZD:])
    z = mean + eps[...] * std
    phi_z = jnp.maximum(
        jnp.dot(z, wpz[...], preferred_element_type=jnp.float32) + bpz[...], 0.0)
    dp = jnp.dot(z, wdec[...], preferred_element_type=jnp.float32)
    g = gxp[...] + jnp.dot(phi_z, wxb[...], preferred_element_type=jnp.float32)
    hh = h[...]
    zg_pre = g[:, :HD] + jnp.dot(hh, whz[...], preferred_element_type=jnp.float32)
    rg_pre = g[:, HD:2 * HD] + jnp.dot(hh, whr[...],
                                       preferred_element_type=jnp.float32)
    xh_pre = g[:, 2 * HD:]
    o_dec[...] = dvv * dp
    o_zg[...] = dvv * zg_pre
    o_rg[...] = dvv * rg_pre
    o_xh[...] = dvv * xh_pre


def _s4_body(rdec, rzg, rrg, rxh, xdec, xzg, xrg, xxh, dv, bdec, bzz, brr,
             h, whh, o_zdec, o_zg, o_xhagg, o_xshh):
    dvv = lax.rsqrt(dv[...] + 1.0)
    o_zdec[...] = dvv * (rdec[0] + rdec[1] + xdec[...]) + bdec[...]
    zg = jax.nn.sigmoid(dvv * (rzg[0] + rzg[1] + xzg[...]) + bzz[...])
    rg = jax.nn.sigmoid(dvv * (rrg[0] + rrg[1] + xrg[...]) + brr[...])
    o_zg[...] = zg
    o_xhagg[...] = dvv * (rxh[0] + rxh[1] + xxh[...])
    o_xshh[...] = dvv * jnp.dot(rg * h[...], whh[...],
                                preferred_element_type=jnp.float32)


def _s5_body(rhh, xhh, dv, xhagg, bxhh, zg, h, o_h):
    ht = jnp.tanh(xhagg[...] + lax.rsqrt(dv[...] + 1.0)
                  * (rhh[0] + rhh[1] + xhh[...]) + bxhh[...])
    zgv = zg[...]
    o_h[...] = zgv * h[...] + (1.0 - zgv) * ht


def _f32(shape):
    return jax.ShapeDtypeStruct(shape, jnp.float32)


def kernel(x, edge_all_list, edge_idx_list, edge_droped_idx_list, params):
    p = params
    ei = edge_idx_list.astype(jnp.int32)

    pad_src = jnp.zeros((EP - E,), jnp.int32)
    pad_dst = jnp.full((EP - E,), N, jnp.int32)
    srcs = [jnp.concatenate([ei[t, 0], pad_src]) for t in range(T)]
    dsts = [jnp.concatenate([ei[t, 1], pad_dst]) for t in range(T)]

    xp = jnp.pad(x, ((0, 0), (0, NP - N), (0, 0))).reshape(T * NP, XD)
    eps_p = jnp.pad(p['eps1'], ((0, NP - N), (0, 0)))

    wenc_t, wenc_b = p['Wenc'][:HD], p['Wenc'][HD:]
    wgt = jnp.concatenate([p['Wxz'][:HD], p['Wxr'][:HD], p['Wxh'][:HD]], 1)
    wxb = jnp.concatenate([p['Wxz'][HD:], p['Wxr'][HD:], p['Wxh'][HD:]], 1)
    wems = jnp.concatenate([p['Wem'], p['Wes']], 1)
    bems = jnp.concatenate([p['bem'], p['bes']])[None, :]
    bzz = (p['bxz'] + p['bhz'])[None, :]
    brr = (p['bxr'] + p['bhr'])[None, :]
    bxhh = (p['bxh'] + p['bhh'])[None, :]
    bpx = p['bpx'][None, :]
    bpz = p['bpz'][None, :]
    benc = p['benc'][None, :]
    bdec = p['bdec'][None, :]

    deg_flat = _deg_dinv(dsts[0], dsts[1], dsts[2])
    deg_b = deg_flat.reshape(T, NP, 128)

    a1_all, gxp_all = pl.pallas_call(
        _pre_body,
        grid=(T * NP // RB,),
        in_specs=[_row_spec(XD), _full_spec((XD, HD)), _full_spec((1, HD)),
                  _full_spec((HD, HD)), _full_spec((HD, 3 * HD))],
        out_specs=[_row_spec(HD), _row_spec(3 * HD)],
        out_shape=[_f32((T * NP, HD)), _f32((T * NP, 3 * HD))],
    )(xp, p['Wpx'], bpx, wenc_t, wgt)
    a1_all = a1_all.reshape(T, NP, HD)
    gxp_all = gxp_all.reshape(T, NP, 3 * HD)

    h = jnp.zeros((NP, HD), jnp.float32)
    zdecs = []
    for t in range(T):
        dv = deg_b[t]
        src, dst = srcs[t], dsts[t]

        xs_enc = pl.pallas_call(
            _s1_body,
            grid=(GRID,),
            in_specs=[_row_spec(HD), _row_spec(HD), _full_spec((HD, HD)),
                      _row_spec(128)],
            out_specs=_row_spec(HD),
            out_shape=_f32((NP, HD)),
        )(a1_all[t], h, wenc_b, dv)

        (r_enc,) = _agg1(xs_enc, src, dst)

        xs_ms = pl.pallas_call(
            _s2_body,
            grid=(GRID,),
            in_specs=[_raw_spec(HD), _row_spec(HD), _row_spec(128),
                      _full_spec((1, HD)), _full_spec((HD, 2 * ZD))],
            out_specs=_row_spec(2 * ZD),
            out_shape=_f32((NP, 2 * ZD)),
        )(r_enc, xs_enc, dv, benc, wems)

        (r_ms,) = _agg1(xs_ms, src, dst)

        xs_dec, xs_zg, xs_rg, xs_xh = pl.pallas_call(
            _s3_body,
            grid=(GRID,),
            in_specs=[_raw_spec(2 * ZD), _row_spec(2 * ZD), _row_spec(128),
                      _full_spec((1, 2 * ZD)), _row_spec(ZD),
                      _full_spec((ZD, HD)), _full_spec((1, HD)),
                      _full_spec((ZD, HD)), _row_spec(3 * HD),
                      _full_spec((HD, 3 * HD)), _row_spec(HD),
                      _full_spec((HD, HD)), _full_spec((HD, HD))],
            out_specs=[_row_spec(HD)] * 4,
            out_shape=[_f32((NP, HD))] * 4,
        )(r_ms, xs_ms, dv, bems, eps_p, p['Wpz'], bpz, p['Wdec'],
          gxp_all[t], wxb, h, p['Whz'], p['Whr'])

        r_dec, r_zg, r_rg, r_xh = _agg4(xs_dec, xs_zg, xs_rg, xs_xh, src, dst)

        zdec_t, zg, xh_agg, xs_hh = pl.pallas_call(
            _s4_body,
            grid=(GRID,),
            in_specs=[_raw_spec(HD)] * 4 + [_row_spec(HD)] * 4
                     + [_row_spec(128), _full_spec((1, HD)),
                        _full_spec((1, HD)), _full_spec((1, HD)),
                        _row_spec(HD), _full_spec((HD, HD))],
            out_specs=[_row_spec(HD)] * 4,
            out_shape=[_f32((NP, HD))] * 4,
        )(r_dec, r_zg, r_rg, r_xh, xs_dec, xs_zg, xs_rg, xs_xh, dv,
          bdec, bzz, brr, h, p['Whh'])

        (r_hh,) = _agg1(xs_hh, src, dst)

        h = pl.pallas_call(
            _s5_body,
            grid=(GRID,),
            in_specs=[_raw_spec(HD), _row_spec(HD), _row_spec(128),
                      _row_spec(HD), _full_spec((1, HD)), _row_spec(HD),
                      _row_spec(HD)],
            out_specs=_row_spec(HD),
            out_shape=_f32((NP, HD)),
        )(r_hh, xs_hh, dv, xh_agg, bxhh, zg, h)

        zdecs.append(zdec_t[:N])

    return jnp.stack(zdecs)


# speedup vs baseline: 6.0676x; 1.0115x over previous
import functools

import jax
import jax.numpy as jnp
from jax import lax
from jax.experimental import pallas as pl
from jax.experimental.pallas import tpu as pltpu
from jax.experimental.pallas import tpu_sc as plsc

N = 10000
T = 3
E = 320000
XD = 128
HD = 128
ZD = 64

NP = 10240
RB = 256
GRID = NP // RB
NC = 2
NS = 16
BPW = 80
NB = NC * NS * BPW
EP = NB * 128
RPT = NP // NS
RPW = NP // (NC * NS)

_mesh = plsc.VectorSubcoreMesh(core_axis_name="c", subcore_axis_name="s")


def _deg_dinv_body(d0, d1, d2, out, dacc, zvec, ones, didx_all, dtile, bbuf):
    c = lax.axis_index("c")
    s = lax.axis_index("s")
    jt = NB // NS
    zero16 = jnp.zeros((16,), jnp.float32)
    one16 = jnp.ones((16,), jnp.float32)
    for i in range(RPT // 16):
        zvec[pl.ds(i * 16, 16)] = zero16
    for i in range(8):
        ones[pl.ds(i * 16, 16)] = one16
    for t, dref in enumerate((d0, d1, d2)):
        pltpu.sync_copy(zvec, dacc.at[pl.ds(s * RPT, RPT)])
        pltpu.sync_copy(dref.at[pl.ds(s * jt, jt)], didx_all)
        plsc.subcore_barrier()

        def hist(j, carry):
            pltpu.sync_copy(ones, dacc.at[didx_all.at[j]], add=True)
            return carry

        lax.fori_loop(0, jt, hist, 0)
        plsc.subcore_barrier()

        row0 = (s * NC + c) * RPW
        pltpu.sync_copy(dacc.at[pl.ds(row0, RPW)], dtile)

        def bc(k, carry):
            y = dtile[pl.ds(k * 16, 16)]
            for l in range(16):
                row = jnp.full((16,), y[l], jnp.float32)
                for c8 in range(8):
                    bbuf[pl.ds((k * 16 + l) * 128 + c8 * 16, 16)] = row
            return carry

        lax.fori_loop(0, RPW // 16, bc, 0)
        pltpu.sync_copy(bbuf,
                        out.at[pl.ds((t * NP + row0) * 128, RPW * 128)])
        plsc.subcore_barrier()


_deg_dinv = functools.partial(
    pl.kernel,
    out_type=jax.ShapeDtypeStruct((T * NP * 128,), jnp.float32),
    mesh=_mesh,
    scratch_types=[
        pltpu.VMEM_SHARED((NP,), jnp.float32),
        pltpu.VMEM((RPT,), jnp.float32),
        pltpu.VMEM((128,), jnp.float32),
        pltpu.VMEM((NB // NS, 128), jnp.int32),
        pltpu.VMEM((RPW,), jnp.float32),
        pltpu.VMEM((RPW * 128,), jnp.float32),
    ],
)(_deg_dinv_body)


CH = 40


def _make_agg(ntab, split_tables):
    jt = (NB // NS) if split_tables else BPW
    tpc = ntab // NC if split_tables else ntab

    def body(*refs):
        tabs = refs[:ntab]
        src = refs[ntab]
        dst = refs[ntab + 1]
        outs = refs[ntab + 2:2 * ntab + 2]
        (acc, sidx_all, didx_all, zbuf, b0, b1,
         g0, g1, s0, s1) = refs[2 * ntab + 2:]
        bufs = (b0, b1)
        gsems = (g0, g1)
        ssems = (s0, s1)
        c = lax.axis_index("c")
        s = lax.axis_index("s")
        zero16 = jnp.zeros((16,), jnp.float32)
        for i in range(8):
            for j in range(8):
                zbuf[i, pl.ds(j * 16, 16)] = zero16
        if split_tables:
            b_lo = s * jt
        else:
            b_lo = (c * NS + s) * jt

        def run_table(tab, flush_dst):
            def gst(j, u):
                pltpu.async_copy(tab.at[sidx_all.at[j]], bufs[u], gsems[u])

            def gwt(u):
                pltpu.make_async_copy(
                    tab.at[sidx_all.at[0]], bufs[u], gsems[u]).wait()

            def sst(j, u):
                pltpu.async_copy(bufs[u], acc.at[didx_all.at[j]], ssems[u],
                                 add=True)

            def swt(u):
                pltpu.make_async_copy(
                    bufs[u], acc.at[didx_all.at[0]], ssems[u]).wait()

            def zb(i, carry):
                pltpu.sync_copy(zbuf, acc.at[pl.ds(s * RPT + i * 8, 8)])
                return carry

            lax.fori_loop(0, RPT // 8, zb, 0)
            plsc.subcore_barrier()

            def chunk(ci, carry):
                pltpu.sync_copy(src.at[pl.ds(b_lo + ci * CH, CH)], sidx_all)
                pltpu.sync_copy(dst.at[pl.ds(b_lo + ci * CH, CH)], didx_all)
                gst(0, 0)

                def pair(jj, carry2):
                    for u in range(2):
                        j = jj * 2 + u
                        gwt(u)
                        sst(j, u)
                        u2 = (u + 1) % 2

                        @pl.when(j >= 1)
                        def _():
                            swt(u2)

                        @pl.when(j + 1 < CH)
                        def _():
                            gst(j + 1, u2)
                    return carry2

                lax.fori_loop(0, CH // 2, pair, 0)
                swt((CH - 1) % 2)
                return carry

            lax.fori_loop(0, jt // CH, chunk, 0)
            plsc.subcore_barrier()
            pltpu.sync_copy(acc.at[pl.ds(s * RPT, RPT)], flush_dst)
            plsc.subcore_barrier()

        if split_tables:
            for k in range(ntab):
                @pl.when(c == k // tpc)
                def _(k=k):
                    run_table(tabs[k], outs[k].at[pl.ds(s * RPT, RPT)])
        else:
            for k in range(ntab):
                run_table(tabs[k], outs[k].at[c, pl.ds(s * RPT, RPT)])

    if split_tables:
        out_t = [jax.ShapeDtypeStruct((NP, 128), jnp.float32)] * ntab
    else:
        out_t = [jax.ShapeDtypeStruct((NC, NP, 128), jnp.float32)] * ntab
    return pl.kernel(
        body,
        out_type=out_t,
        mesh=_mesh,
        scratch_types=[
            pltpu.VMEM_SHARED((NP, 128), jnp.float32),
            pltpu.VMEM((CH, 128), jnp.int32),
            pltpu.VMEM((CH, 128), jnp.int32),
            pltpu.VMEM((8, 128), jnp.float32),
            pltpu.VMEM((128, 128), jnp.float32),
            pltpu.VMEM((128, 128), jnp.float32),
            pltpu.SemaphoreType.DMA,
            pltpu.SemaphoreType.DMA,
            pltpu.SemaphoreType.DMA,
            pltpu.SemaphoreType.DMA,
        ],
    )


_agg1 = _make_agg(1, False)
_agg4 = _make_agg(4, True)


def _row_spec(cols):
    return pl.BlockSpec((RB, cols), lambda i: (i, 0))


def _raw_spec(cols):
    return pl.BlockSpec((NC, RB, cols), lambda i: (0, i, 0))


def _full_spec(shape):
    nd = len(shape)
    return pl.BlockSpec(shape, lambda i: (0,) * nd)


def _pre_body(x_ref, wpx, bpx, wet, wgt, o_a1, o_gxp):
    phi = jnp.maximum(
        jnp.dot(x_ref[...], wpx[...], preferred_element_type=jnp.float32)
        + bpx[...], 0.0)
    o_a1[...] = jnp.dot(phi, wet[...], preferred_element_type=jnp.float32)
    o_gxp[...] = jnp.dot(phi, wgt[...], preferred_element_type=jnp.float32)


def _s1_body(a1, h, wencb, dv, o):
    dvv = lax.rsqrt(dv[...] + 1.0)
    o[...] = dvv * (
        a1[...] + jnp.dot(h[...], wencb[...], preferred_element_type=jnp.float32))


def _s2_body(renc, xenc, dv, benc, wems, o):
    dvv = lax.rsqrt(dv[...] + 1.0)
    enc = jnp.maximum(
        dvv * (renc[0] + renc[1] + xenc[...]) + benc[...], 0.0)
    o[...] = dvv * jnp.dot(enc, wems[...], preferred_element_type=jnp.float32)


def _softplus(x):
    return jnp.maximum(x, 0.0) + jnp.log1p(jnp.exp(-jnp.abs(x)))


def _s3_body(rms, xms, dv, bems, eps, wpz, bpz, wdec, gxp, wxb, h, whz, whr,
             o_dec, o_zg, o_rg, o_xh):
    dvv = lax.rsqrt(dv[...] + 1.0)
    agg = dvv * (rms[0] + rms[1] + xms[...]) + bems[...]
    mean = agg[:, :ZD]
    std = _softplus(agg[:, ZD:])
    z = mean + eps[...] * std
    phi_z = jnp.maximum(
        jnp.dot(z, wpz[...], preferred_element_type=jnp.float32) + bpz[...], 0.0)
    dp = jnp.dot(z, wdec[...], preferred_element_type=jnp.float32)
    g = gxp[...] + jnp.dot(phi_z, wxb[...], preferred_element_type=jnp.float32)
    hh = h[...]
    zg_pre = g[:, :HD] + jnp.dot(hh, whz[...], preferred_element_type=jnp.float32)
    rg_pre = g[:, HD:2 * HD] + jnp.dot(hh, whr[...],
                                       preferred_element_type=jnp.float32)
    xh_pre = g[:, 2 * HD:]
    o_dec[...] = dvv * dp
    o_zg[...] = dvv * zg_pre
    o_rg[...] = dvv * rg_pre
    o_xh[...] = dvv * xh_pre


def _s4_body(rdec, rzg, rrg, rxh, xdec, xzg, xrg, xxh, dv, bdec, bzz, brr,
             h, whh, o_zdec, o_zg, o_xhagg, o_xshh):
    dvv = lax.rsqrt(dv[...] + 1.0)
    o_zdec[...] = dvv * (rdec[...] + xdec[...]) + bdec[...]
    zg = jax.nn.sigmoid(dvv * (rzg[...] + xzg[...]) + bzz[...])
    rg = jax.nn.sigmoid(dvv * (rrg[...] + xrg[...]) + brr[...])
    o_zg[...] = zg
    o_xhagg[...] = dvv * (rxh[...] + xxh[...])
    o_xshh[...] = dvv * jnp.dot(rg * h[...], whh[...],
                                preferred_element_type=jnp.float32)


def _s5_body(rhh, xhh, dv, xhagg, bxhh, zg, h, o_h):
    ht = jnp.tanh(xhagg[...] + lax.rsqrt(dv[...] + 1.0)
                  * (rhh[0] + rhh[1] + xhh[...]) + bxhh[...])
    zgv = zg[...]
    o_h[...] = zgv * h[...] + (1.0 - zgv) * ht


def _f32(shape):
    return jax.ShapeDtypeStruct(shape, jnp.float32)


def kernel(x, edge_all_list, edge_idx_list, edge_droped_idx_list, params):
    p = params
    ei = edge_idx_list.astype(jnp.int32)

    pad_src = jnp.zeros((EP - E,), jnp.int32)
    pad_dst = jnp.full((EP - E,), N, jnp.int32)
    srcs = [jnp.concatenate([ei[t, 0], pad_src]).reshape(NB, 128)
            for t in range(T)]
    dsts = [jnp.concatenate([ei[t, 1], pad_dst]).reshape(NB, 128)
            for t in range(T)]

    xp = jnp.pad(x, ((0, 0), (0, NP - N), (0, 0))).reshape(T * NP, XD)
    eps_p = jnp.pad(p['eps1'], ((0, NP - N), (0, 0)))

    wenc_t, wenc_b = p['Wenc'][:HD], p['Wenc'][HD:]
    wgt = jnp.concatenate([p['Wxz'][:HD], p['Wxr'][:HD], p['Wxh'][:HD]], 1)
    wxb = jnp.concatenate([p['Wxz'][HD:], p['Wxr'][HD:], p['Wxh'][HD:]], 1)
    wems = jnp.concatenate([p['Wem'], p['Wes']], 1)
    bems = jnp.concatenate([p['bem'], p['bes']])[None, :]
    bzz = (p['bxz'] + p['bhz'])[None, :]
    brr = (p['bxr'] + p['bhr'])[None, :]
    bxhh = (p['bxh'] + p['bhh'])[None, :]
    bpx = p['bpx'][None, :]
    bpz = p['bpz'][None, :]
    benc = p['benc'][None, :]
    bdec = p['bdec'][None, :]

    deg_flat = _deg_dinv(dsts[0], dsts[1], dsts[2])
    deg_b = deg_flat.reshape(T, NP, 128)

    a1_all, gxp_all = pl.pallas_call(
        _pre_body,
        grid=(T * NP // RB,),
        in_specs=[_row_spec(XD), _full_spec((XD, HD)), _full_spec((1, HD)),
                  _full_spec((HD, HD)), _full_spec((HD, 3 * HD))],
        out_specs=[_row_spec(HD), _row_spec(3 * HD)],
        out_shape=[_f32((T * NP, HD)), _f32((T * NP, 3 * HD))],
    )(xp, p['Wpx'], bpx, wenc_t, wgt)
    a1_all = a1_all.reshape(T, NP, HD)
    gxp_all = gxp_all.reshape(T, NP, 3 * HD)

    h = jnp.zeros((NP, HD), jnp.float32)
    zdecs = []
    for t in range(T):
        dv = deg_b[t]
        src, dst = srcs[t], dsts[t]

        xs_enc = pl.pallas_call(
            _s1_body,
            grid=(GRID,),
            in_specs=[_row_spec(HD), _row_spec(HD), _full_spec((HD, HD)),
                      _row_spec(128)],
            out_specs=_row_spec(HD),
            out_shape=_f32((NP, HD)),
        )(a1_all[t], h, wenc_b, dv)

        (r_enc,) = _agg1(xs_enc, src, dst)

        xs_ms = pl.pallas_call(
            _s2_body,
            grid=(GRID,),
            in_specs=[_raw_spec(HD), _row_spec(HD), _row_spec(128),
                      _full_spec((1, HD)), _full_spec((HD, 2 * ZD))],
            out_specs=_row_spec(2 * ZD),
            out_shape=_f32((NP, 2 * ZD)),
        )(r_enc, xs_enc, dv, benc, wems)

        (r_ms,) = _agg1(xs_ms, src, dst)

        xs_dec, xs_zg, xs_rg, xs_xh = pl.pallas_call(
            _s3_body,
            grid=(GRID,),
            in_specs=[_raw_spec(2 * ZD), _row_spec(2 * ZD), _row_spec(128),
                      _full_spec((1, 2 * ZD)), _row_spec(ZD),
                      _full_spec((ZD, HD)), _full_spec((1, HD)),
                      _full_spec((ZD, HD)), _row_spec(3 * HD),
                      _full_spec((HD, 3 * HD)), _row_spec(HD),
                      _full_spec((HD, HD)), _full_spec((HD, HD))],
            out_specs=[_row_spec(HD)] * 4,
            out_shape=[_f32((NP, HD))] * 4,
        )(r_ms, xs_ms, dv, bems, eps_p, p['Wpz'], bpz, p['Wdec'],
          gxp_all[t], wxb, h, p['Whz'], p['Whr'])

        r_dec, r_zg, r_rg, r_xh = _agg4(xs_dec, xs_zg, xs_rg, xs_xh, src, dst)

        zdec_t, zg, xh_agg, xs_hh = pl.pallas_call(
            _s4_body,
            grid=(GRID,),
            in_specs=[_row_spec(HD)] * 4 + [_row_spec(HD)] * 4
                     + [_row_spec(128), _full_spec((1, HD)),
                        _full_spec((1, HD)), _full_spec((1, HD)),
                        _row_spec(HD), _full_spec((HD, HD))],
            out_specs=[_row_spec(HD)] * 4,
            out_shape=[_f32((NP, HD))] * 4,
        )(r_dec, r_zg, r_rg, r_xh, xs_dec, xs_zg, xs_rg, xs_xh, dv,
          bdec, bzz, brr, h, p['Whh'])

        (r_hh,) = _agg1(xs_hh, src, dst)

        h = pl.pallas_call(
            _s5_body,
            grid=(GRID,),
            in_specs=[_raw_spec(HD), _row_spec(HD), _row_spec(128),
                      _row_spec(HD), _full_spec((1, HD)), _row_spec(HD),
                      _row_spec(HD)],
            out_specs=_row_spec(HD),
            out_shape=_f32((NP, HD)),
        )(r_hh, xs_hh, dv, xh_agg, bxhh, zg, h)

        zdecs.append(zdec_t[:N])

    return jnp.stack(zdecs)


# speedup vs baseline: 6.3520x; 1.0469x over previous
import functools

import jax
import jax.numpy as jnp
from jax import lax
from jax.experimental import pallas as pl
from jax.experimental.pallas import tpu as pltpu
from jax.experimental.pallas import tpu_sc as plsc

N = 10000
T = 3
E = 320000
XD = 128
HD = 128
ZD = 64

NP = 10240
RB = 256
GRID = NP // RB
NC = 2
NS = 16
BPW = 80
NB = NC * NS * BPW
EP = NB * 128
RPT = NP // NS
RPW = NP // (NC * NS)

_mesh = plsc.VectorSubcoreMesh(core_axis_name="c", subcore_axis_name="s")


def _deg_dinv_body(d0, d1, d2, out, dacc, zvec, ones, didx_all, dtile, bbuf):
    c = lax.axis_index("c")
    s = lax.axis_index("s")
    jt = NB // NS
    zero16 = jnp.zeros((16,), jnp.float32)
    one16 = jnp.ones((16,), jnp.float32)
    for i in range(RPT // 16):
        zvec[pl.ds(i * 16, 16)] = zero16
    for i in range(8):
        ones[pl.ds(i * 16, 16)] = one16
    for t, dref in enumerate((d0, d1, d2)):
        pltpu.sync_copy(zvec, dacc.at[pl.ds(s * RPT, RPT)])
        pltpu.sync_copy(dref.at[pl.ds(s * jt, jt)], didx_all)
        plsc.subcore_barrier()

        def hist(j, carry):
            pltpu.sync_copy(ones, dacc.at[didx_all.at[j]], add=True)
            return carry

        lax.fori_loop(0, jt, hist, 0)
        plsc.subcore_barrier()

        row0 = (s * NC + c) * RPW
        pltpu.sync_copy(dacc.at[pl.ds(row0, RPW)], dtile)

        def bc(k, carry):
            y = dtile[pl.ds(k * 16, 16)]
            for l in range(16):
                row = jnp.full((16,), y[l], jnp.float32)
                for c8 in range(8):
                    bbuf[pl.ds((k * 16 + l) * 128 + c8 * 16, 16)] = row
            return carry

        lax.fori_loop(0, RPW // 16, bc, 0)
        pltpu.sync_copy(bbuf,
                        out.at[pl.ds((t * NP + row0) * 128, RPW * 128)])
        plsc.subcore_barrier()


_deg_dinv = functools.partial(
    pl.kernel,
    out_type=jax.ShapeDtypeStruct((T * NP * 128,), jnp.float32),
    mesh=_mesh,
    scratch_types=[
        pltpu.VMEM_SHARED((NP,), jnp.float32),
        pltpu.VMEM((RPT,), jnp.float32),
        pltpu.VMEM((128,), jnp.float32),
        pltpu.VMEM((NB // NS, 128), jnp.int32),
        pltpu.VMEM((RPW,), jnp.float32),
        pltpu.VMEM((RPW * 128,), jnp.float32),
    ],
)(_deg_dinv_body)


CH = 40


def _make_agg(ntab, split_tables):
    jt = (NB // NS) if split_tables else BPW
    tpc = ntab // NC if split_tables else ntab

    def body(*refs):
        tabs = refs[:ntab]
        src = refs[ntab]
        dst = refs[ntab + 1]
        outs = refs[ntab + 2:2 * ntab + 2]
        (acc, sidx_all, didx_all, zbuf, b0, b1,
         g0, g1, s0, s1) = refs[2 * ntab + 2:]
        bufs = (b0, b1)
        gsems = (g0, g1)
        ssems = (s0, s1)
        c = lax.axis_index("c")
        s = lax.axis_index("s")
        zero16 = jnp.zeros((16,), jnp.float32)
        for i in range(16):
            for j in range(8):
                zbuf[i, pl.ds(j * 16, 16)] = zero16
        if split_tables:
            b_lo = s * jt
        else:
            b_lo = (c * NS + s) * jt

        def run_table(tab, flush_dst):
            def gst(j, u):
                pltpu.async_copy(tab.at[sidx_all.at[j]], bufs[u], gsems[u])

            def gwt(u):
                pltpu.make_async_copy(
                    tab.at[sidx_all.at[0]], bufs[u], gsems[u]).wait()

            def sst(j, u):
                pltpu.async_copy(bufs[u], acc.at[didx_all.at[j]], ssems[u],
                                 add=True)

            def swt(u):
                pltpu.make_async_copy(
                    bufs[u], acc.at[didx_all.at[0]], ssems[u]).wait()

            def zb(i, carry):
                pltpu.sync_copy(zbuf, acc.at[pl.ds(s * RPT + i * 16, 16)])
                return carry

            lax.fori_loop(0, RPT // 16, zb, 0)
            plsc.subcore_barrier()

            def chunk(ci, carry):
                pltpu.sync_copy(src.at[pl.ds(b_lo + ci * CH, CH)], sidx_all)
                pltpu.sync_copy(dst.at[pl.ds(b_lo + ci * CH, CH)], didx_all)
                gst(0, 0)

                def pair(jj, carry2):
                    for u in range(2):
                        j = jj * 2 + u
                        u2 = (u + 1) % 2

                        @pl.when(j >= 1)
                        def _():
                            swt(u2)

                        @pl.when(j + 1 < CH)
                        def _():
                            gst(j + 1, u2)

                        gwt(u)
                        sst(j, u)
                    return carry2

                lax.fori_loop(0, CH // 2, pair, 0)
                swt((CH - 1) % 2)
                return carry

            lax.fori_loop(0, jt // CH, chunk, 0)
            plsc.subcore_barrier()
            pltpu.sync_copy(acc.at[pl.ds(s * RPT, RPT)], flush_dst)
            plsc.subcore_barrier()

        if split_tables:
            for k in range(ntab):
                @pl.when(c == k // tpc)
                def _(k=k):
                    run_table(tabs[k], outs[k].at[pl.ds(s * RPT, RPT)])
        else:
            for k in range(ntab):
                run_table(tabs[k], outs[k].at[c, pl.ds(s * RPT, RPT)])

    if split_tables:
        out_t = [jax.ShapeDtypeStruct((NP, 128), jnp.float32)] * ntab
    else:
        out_t = [jax.ShapeDtypeStruct((NC, NP, 128), jnp.float32)] * ntab
    return pl.kernel(
        body,
        out_type=out_t,
        mesh=_mesh,
        scratch_types=[
            pltpu.VMEM_SHARED((NP, 128), jnp.float32),
            pltpu.VMEM((CH, 128), jnp.int32),
            pltpu.VMEM((CH, 128), jnp.int32),
            pltpu.VMEM((16, 128), jnp.float32),
            pltpu.VMEM((128, 128), jnp.float32),
            pltpu.VMEM((128, 128), jnp.float32),
            pltpu.SemaphoreType.DMA,
            pltpu.SemaphoreType.DMA,
            pltpu.SemaphoreType.DMA,
            pltpu.SemaphoreType.DMA,
        ],
    )


_agg1 = _make_agg(1, False)
_agg4 = _make_agg(4, True)


def _row_spec(cols):
    return pl.BlockSpec((RB, cols), lambda i: (i, 0))


def _raw_spec(cols):
    return pl.BlockSpec((NC, RB, cols), lambda i: (0, i, 0))


def _full_spec(shape):
    nd = len(shape)
    return pl.BlockSpec(shape, lambda i: (0,) * nd)


def _pre_body(x_ref, wpx, bpx, wet, wgt, o_a1, o_gxp):
    phi = jnp.maximum(
        jnp.dot(x_ref[...], wpx[...], preferred_element_type=jnp.float32)
        + bpx[...], 0.0)
    o_a1[...] = jnp.dot(phi, wet[...], preferred_element_type=jnp.float32)
    o_gxp[...] = jnp.dot(phi, wgt[...], preferred_element_type=jnp.float32)


def _s1_body(a1, h, wencb, dv, o):
    dvv = lax.rsqrt(dv[...] + 1.0)
    o[...] = dvv * (
        a1[...] + jnp.dot(h[...], wencb[...], preferred_element_type=jnp.float32))


def _s2_body(renc, xenc, dv, benc, wems, o):
    dvv = lax.rsqrt(dv[...] + 1.0)
    enc = jnp.maximum(
        dvv * (renc[0] + renc[1] + xenc[...]) + benc[...], 0.0)
    o[...] = dvv * jnp.dot(enc, wems[...], preferred_element_type=jnp.float32)


def _softplus(x):
    return jnp.maximum(x, 0.0) + jnp.log1p(jnp.exp(-jnp.abs(x)))


def _s3_body(rms, xms, dv, bems, eps, wpz, bpz, wdec, gxp, wxb, h, whz, whr,
             o_dec, o_zg, o_rg, o_xh):
    dvv = lax.rsqrt(dv[...] + 1.0)
    agg = dvv * (rms[0] + rms[1] + xms[...]) + bems[...]
    mean = agg[:, :ZD]
    std = _softplus(agg[:, ZD:])
    z = mean + eps[...] * std
    phi_z = jnp.maximum(
        jnp.dot(z, wpz[...], preferred_element_type=jnp.float32) + bpz[...], 0.0)
    dp = jnp.dot(z, wdec[...], preferred_element_type=jnp.float32)
    g = gxp[...] + jnp.dot(phi_z, wxb[...], preferred_element_type=jnp.float32)
    hh = h[...]
    zg_pre = g[:, :HD] + jnp.dot(hh, whz[...], preferred_element_type=jnp.float32)
    rg_pre = g[:, HD:2 * HD] + jnp.dot(hh, whr[...],
                                       preferred_element_type=jnp.float32)
    xh_pre = g[:, 2 * HD:]
    o_dec[...] = dvv * dp
    o_zg[...] = dvv * zg_pre
    o_rg[...] = dvv * rg_pre
    o_xh[...] = dvv * xh_pre


def _s4_body(rdec, rzg, rrg, rxh, xdec, xzg, xrg, xxh, dv, bdec, bzz, brr,
             h, whh, o_zdec, o_zg, o_xhagg, o_xshh):
    dvv = lax.rsqrt(dv[...] + 1.0)
    o_zdec[...] = dvv * (rdec[...] + xdec[...]) + bdec[...]
    zg = jax.nn.sigmoid(dvv * (rzg[...] + xzg[...]) + bzz[...])
    rg = jax.nn.sigmoid(dvv * (rrg[...] + xrg[...]) + brr[...])
    o_zg[...] = zg
    o_xhagg[...] = dvv * (rxh[...] + xxh[...])
    o_xshh[...] = dvv * jnp.dot(rg * h[...], whh[...],
                                preferred_element_type=jnp.float32)


def _s5_body(rhh, xhh, dv, xhagg, bxhh, zg, h, o_h):
    ht = jnp.tanh(xhagg[...] + lax.rsqrt(dv[...] + 1.0)
                  * (rhh[0] + rhh[1] + xhh[...]) + bxhh[...])
    zgv = zg[...]
    o_h[...] = zgv * h[...] + (1.0 - zgv) * ht


def _f32(shape):
    return jax.ShapeDtypeStruct(shape, jnp.float32)


def kernel(x, edge_all_list, edge_idx_list, edge_droped_idx_list, params):
    p = params
    ei = edge_idx_list.astype(jnp.int32)

    pad_src = jnp.zeros((EP - E,), jnp.int32)
    pad_dst = jnp.full((EP - E,), N, jnp.int32)
    srcs = [jnp.concatenate([ei[t, 0], pad_src]).reshape(NB, 128)
            for t in range(T)]
    dsts = [jnp.concatenate([ei[t, 1], pad_dst]).reshape(NB, 128)
            for t in range(T)]

    xp = jnp.pad(x, ((0, 0), (0, NP - N), (0, 0))).reshape(T * NP, XD)
    eps_p = jnp.pad(p['eps1'], ((0, NP - N), (0, 0)))

    wenc_t, wenc_b = p['Wenc'][:HD], p['Wenc'][HD:]
    wgt = jnp.concatenate([p['Wxz'][:HD], p['Wxr'][:HD], p['Wxh'][:HD]], 1)
    wxb = jnp.concatenate([p['Wxz'][HD:], p['Wxr'][HD:], p['Wxh'][HD:]], 1)
    wems = jnp.concatenate([p['Wem'], p['Wes']], 1)
    bems = jnp.concatenate([p['bem'], p['bes']])[None, :]
    bzz = (p['bxz'] + p['bhz'])[None, :]
    brr = (p['bxr'] + p['bhr'])[None, :]
    bxhh = (p['bxh'] + p['bhh'])[None, :]
    bpx = p['bpx'][None, :]
    bpz = p['bpz'][None, :]
    benc = p['benc'][None, :]
    bdec = p['bdec'][None, :]

    deg_flat = _deg_dinv(dsts[0], dsts[1], dsts[2])
    deg_b = deg_flat.reshape(T, NP, 128)

    a1_all, gxp_all = pl.pallas_call(
        _pre_body,
        grid=(T * NP // RB,),
        in_specs=[_row_spec(XD), _full_spec((XD, HD)), _full_spec((1, HD)),
                  _full_spec((HD, HD)), _full_spec((HD, 3 * HD))],
        out_specs=[_row_spec(HD), _row_spec(3 * HD)],
        out_shape=[_f32((T * NP, HD)), _f32((T * NP, 3 * HD))],
    )(xp, p['Wpx'], bpx, wenc_t, wgt)
    a1_all = a1_all.reshape(T, NP, HD)
    gxp_all = gxp_all.reshape(T, NP, 3 * HD)

    h = jnp.zeros((NP, HD), jnp.float32)
    zdecs = []
    for t in range(T):
        dv = deg_b[t]
        src, dst = srcs[t], dsts[t]

        xs_enc = pl.pallas_call(
            _s1_body,
            grid=(GRID,),
            in_specs=[_row_spec(HD), _row_spec(HD), _full_spec((HD, HD)),
                      _row_spec(128)],
            out_specs=_row_spec(HD),
            out_shape=_f32((NP, HD)),
        )(a1_all[t], h, wenc_b, dv)

        (r_enc,) = _agg1(xs_enc, src, dst)

        xs_ms = pl.pallas_call(
            _s2_body,
            grid=(GRID,),
            in_specs=[_raw_spec(HD), _row_spec(HD), _row_spec(128),
                      _full_spec((1, HD)), _full_spec((HD, 2 * ZD))],
            out_specs=_row_spec(2 * ZD),
            out_shape=_f32((NP, 2 * ZD)),
        )(r_enc, xs_enc, dv, benc, wems)

        (r_ms,) = _agg1(xs_ms, src, dst)

        xs_dec, xs_zg, xs_rg, xs_xh = pl.pallas_call(
            _s3_body,
            grid=(GRID,),
            in_specs=[_raw_spec(2 * ZD), _row_spec(2 * ZD), _row_spec(128),
                      _full_spec((1, 2 * ZD)), _row_spec(ZD),
                      _full_spec((ZD, HD)), _full_spec((1, HD)),
                      _full_spec((ZD, HD)), _row_spec(3 * HD),
                      _full_spec((HD, 3 * HD)), _row_spec(HD),
                      _full_spec((HD, HD)), _full_spec((HD, HD))],
            out_specs=[_row_spec(HD)] * 4,
            out_shape=[_f32((NP, HD))] * 4,
        )(r_ms, xs_ms, dv, bems, eps_p, p['Wpz'], bpz, p['Wdec'],
          gxp_all[t], wxb, h, p['Whz'], p['Whr'])

        r_dec, r_zg, r_rg, r_xh = _agg4(xs_dec, xs_zg, xs_rg, xs_xh, src, dst)

        zdec_t, zg, xh_agg, xs_hh = pl.pallas_call(
            _s4_body,
            grid=(GRID,),
            in_specs=[_row_spec(HD)] * 4 + [_row_spec(HD)] * 4
                     + [_row_spec(128), _full_spec((1, HD)),
                        _full_spec((1, HD)), _full_spec((1, HD)),
                        _row_spec(HD), _full_spec((HD, HD))],
            out_specs=[_row_spec(HD)] * 4,
            out_shape=[_f32((NP, HD))] * 4,
        )(r_dec, r_zg, r_rg, r_xh, xs_dec, xs_zg, xs_rg, xs_xh, dv,
          bdec, bzz, brr, h, p['Whh'])

        (r_hh,) = _agg1(xs_hh, src, dst)

        h = pl.pallas_call(
            _s5_body,
            grid=(GRID,),
            in_specs=[_raw_spec(HD), _row_spec(HD), _row_spec(128),
                      _row_spec(HD), _full_spec((1, HD)), _row_spec(HD),
                      _row_spec(HD)],
            out_specs=_row_spec(HD),
            out_shape=_f32((NP, HD)),
        )(r_hh, xs_hh, dv, xh_agg, bxhh, zg, h)

        zdecs.append(zdec_t[:N])

    return jnp.stack(zdecs)


# speedup vs baseline: 12.5734x; 1.9795x over previous
import functools

import jax
import jax.numpy as jnp
from jax import lax
from jax.experimental import pallas as pl
from jax.experimental.pallas import tpu as pltpu
from jax.experimental.pallas import tpu_sc as plsc

N = 10000
T = 3
E = 320000
XD = 128
HD = 128
ZD = 64

NP = 10240
RB = 256
GRID = NP // RB
NC = 2
NS = 16
BPW = 80
NB = NC * NS * BPW
EP = NB * 128
RPT = NP // NS
RPW = NP // (NC * NS)

_mesh = plsc.VectorSubcoreMesh(core_axis_name="c", subcore_axis_name="s")


def _deg_dinv_body(d0, d1, d2, out, dacc, zvec, ones, didx_all, dtile, bbuf):
    c = lax.axis_index("c")
    s = lax.axis_index("s")
    jt = NB // NS
    zero16 = jnp.zeros((16,), jnp.float32)
    one16 = jnp.ones((16,), jnp.float32)
    for i in range(RPT // 16):
        zvec[pl.ds(i * 16, 16)] = zero16
    for i in range(8):
        ones[pl.ds(i * 16, 16)] = one16
    for t, dref in enumerate((d0, d1, d2)):
        pltpu.sync_copy(zvec, dacc.at[pl.ds(s * RPT, RPT)])
        pltpu.sync_copy(dref.at[pl.ds(s * jt, jt)], didx_all)
        plsc.subcore_barrier()

        def hist(j, carry):
            pltpu.sync_copy(ones, dacc.at[didx_all.at[j]], add=True)
            return carry

        lax.fori_loop(0, jt, hist, 0)
        plsc.subcore_barrier()

        row0 = (s * NC + c) * RPW
        pltpu.sync_copy(dacc.at[pl.ds(row0, RPW)], dtile)

        def bc(k, carry):
            y = dtile[pl.ds(k * 16, 16)]
            for l in range(16):
                row = jnp.full((16,), y[l], jnp.float32)
                for c8 in range(8):
                    bbuf[pl.ds((k * 16 + l) * 128 + c8 * 16, 16)] = row
            return carry

        lax.fori_loop(0, RPW // 16, bc, 0)
        pltpu.sync_copy(bbuf,
                        out.at[pl.ds((t * NP + row0) * 128, RPW * 128)])
        plsc.subcore_barrier()


_deg_dinv = functools.partial(
    pl.kernel,
    out_type=jax.ShapeDtypeStruct((T * NP * 128,), jnp.float32),
    mesh=_mesh,
    scratch_types=[
        pltpu.VMEM_SHARED((NP,), jnp.float32),
        pltpu.VMEM((RPT,), jnp.float32),
        pltpu.VMEM((128,), jnp.float32),
        pltpu.VMEM((NB // NS, 128), jnp.int32),
        pltpu.VMEM((RPW,), jnp.float32),
        pltpu.VMEM((RPW * 128,), jnp.float32),
    ],
)(_deg_dinv_body)


CH = 40


def _make_agg(ntab, split_tables):
    jt = (NB // NS) if split_tables else BPW
    tpc = ntab // NC if split_tables else ntab

    def body(*refs):
        tabs = refs[:ntab]
        src = refs[ntab]
        dst = refs[ntab + 1]
        outs = refs[ntab + 2:2 * ntab + 2]
        (tbl, acc, sidx_all, didx_all, zbuf, b0, b1,
         g0, g1, s0, s1) = refs[2 * ntab + 2:]
        bufs = (b0, b1)
        gsems = (g0, g1)
        ssems = (s0, s1)
        c = lax.axis_index("c")
        s = lax.axis_index("s")
        zero16 = jnp.zeros((16,), jnp.float32)
        for i in range(16):
            for j in range(4):
                zbuf[i, pl.ds(j * 16, 16)] = zero16
        if split_tables:
            b_lo = s * jt
        else:
            b_lo = (c * NS + s) * jt

        def run_half(tab, hh, flush_dst):
            pltpu.sync_copy(tab.at[hh, pl.ds(s * RPT, RPT)],
                            tbl.at[pl.ds(s * RPT, RPT)])

            def gst(j, u):
                pltpu.async_copy(tbl.at[sidx_all.at[j]], bufs[u], gsems[u])

            def gwt(u):
                pltpu.make_async_copy(
                    tbl.at[sidx_all.at[0]], bufs[u], gsems[u]).wait()

            def sst(j, u):
                pltpu.async_copy(bufs[u], acc.at[didx_all.at[j]], ssems[u],
                                 add=True)

            def swt(u):
                pltpu.make_async_copy(
                    bufs[u], acc.at[didx_all.at[0]], ssems[u]).wait()

            def zb(i, carry):
                pltpu.sync_copy(zbuf, acc.at[pl.ds(s * RPT + i * 16, 16)])
                return carry

            lax.fori_loop(0, RPT // 16, zb, 0)
            plsc.subcore_barrier()

            def chunk(ci, carry):
                pltpu.sync_copy(src.at[pl.ds(b_lo + ci * CH, CH)], sidx_all)
                pltpu.sync_copy(dst.at[pl.ds(b_lo + ci * CH, CH)], didx_all)
                gst(0, 0)

                def pair(jj, carry2):
                    for u in range(2):
                        j = jj * 2 + u
                        u2 = (u + 1) % 2

                        @pl.when(j >= 1)
                        def _():
                            swt(u2)

                        @pl.when(j + 1 < CH)
                        def _():
                            gst(j + 1, u2)

                        gwt(u)
                        sst(j, u)
                    return carry2

                lax.fori_loop(0, CH // 2, pair, 0)
                swt((CH - 1) % 2)
                return carry

            lax.fori_loop(0, jt // CH, chunk, 0)
            plsc.subcore_barrier()
            pltpu.sync_copy(acc.at[pl.ds(s * RPT, RPT)], flush_dst)
            plsc.subcore_barrier()

        if split_tables:
            for k in range(ntab):
                @pl.when(c == k // tpc)
                def _(k=k):
                    for hh in range(2):
                        run_half(tabs[k], hh,
                                 outs[k].at[hh, pl.ds(s * RPT, RPT)])
        else:
            for k in range(ntab):
                for hh in range(2):
                    run_half(tabs[k], hh,
                             outs[k].at[c, hh, pl.ds(s * RPT, RPT)])

    if split_tables:
        out_t = [jax.ShapeDtypeStruct((2, NP, 64), jnp.float32)] * ntab
    else:
        out_t = [jax.ShapeDtypeStruct((NC, 2, NP, 64), jnp.float32)] * ntab
    return pl.kernel(
        body,
        out_type=out_t,
        mesh=_mesh,
        scratch_types=[
            pltpu.VMEM_SHARED((NP, 64), jnp.float32),
            pltpu.VMEM_SHARED((NP, 64), jnp.float32),
            pltpu.VMEM((CH, 128), jnp.int32),
            pltpu.VMEM((CH, 128), jnp.int32),
            pltpu.VMEM((16, 64), jnp.float32),
            pltpu.VMEM((128, 64), jnp.float32),
            pltpu.VMEM((128, 64), jnp.float32),
            pltpu.SemaphoreType.DMA,
            pltpu.SemaphoreType.DMA,
            pltpu.SemaphoreType.DMA,
            pltpu.SemaphoreType.DMA,
        ],
    )


_agg1 = _make_agg(1, False)
_agg4 = _make_agg(4, True)


def _row_spec(cols):
    return pl.BlockSpec((RB, cols), lambda i: (i, 0))


def _half_spec():
    return pl.BlockSpec((2, RB, 64), lambda i: (0, i, 0))


def _rawh_spec():
    return pl.BlockSpec((NC, 2, RB, 64), lambda i: (0, 0, i, 0))


def _split2(v):
    return jnp.stack([v[:, :64], v[:, 64:]])


def _cat2(r):
    return jnp.concatenate([r[0], r[1]], axis=1)


def _catsum(r):
    return jnp.concatenate([r[0, 0] + r[1, 0], r[0, 1] + r[1, 1]], axis=1)


def _full_spec(shape):
    nd = len(shape)
    return pl.BlockSpec(shape, lambda i: (0,) * nd)


def _pre_body(x_ref, wpx, bpx, wet, wgt, o_a1, o_gxp):
    phi = jnp.maximum(
        jnp.dot(x_ref[...], wpx[...], preferred_element_type=jnp.float32)
        + bpx[...], 0.0)
    o_a1[...] = jnp.dot(phi, wet[...], preferred_element_type=jnp.float32)
    o_gxp[...] = jnp.dot(phi, wgt[...], preferred_element_type=jnp.float32)


def _s1_body(a1, h, wencb, dv, o):
    dvv = lax.rsqrt(dv[...] + 1.0)
    o[...] = _split2(dvv * (
        a1[...] + jnp.dot(h[...], wencb[...],
                          preferred_element_type=jnp.float32)))


def _s2_body(renc, xenc, dv, benc, wems, o):
    dvv = lax.rsqrt(dv[...] + 1.0)
    enc = jnp.maximum(
        dvv * (_catsum(renc[...]) + _cat2(xenc[...])) + benc[...], 0.0)
    o[...] = _split2(
        dvv * jnp.dot(enc, wems[...], preferred_element_type=jnp.float32))


def _softplus(x):
    return jnp.maximum(x, 0.0) + jnp.log1p(jnp.exp(-jnp.abs(x)))


def _s3_body(rms, xms, dv, bems, eps, wpz, bpz, wdec, gxp, wxb, h, whz, whr,
             o_dec, o_zg, o_rg, o_xh):
    dvv = lax.rsqrt(dv[...] + 1.0)
    agg = dvv * (_catsum(rms[...]) + _cat2(xms[...])) + bems[...]
    mean = agg[:, :ZD]
    std = _softplus(agg[:, ZD:])
    z = mean + eps[...] * std
    phi_z = jnp.maximum(
        jnp.dot(z, wpz[...], preferred_element_type=jnp.float32) + bpz[...], 0.0)
    dp = jnp.dot(z, wdec[...], preferred_element_type=jnp.float32)
    g = gxp[...] + jnp.dot(phi_z, wxb[...], preferred_element_type=jnp.float32)
    hh = h[...]
    zg_pre = g[:, :HD] + jnp.dot(hh, whz[...], preferred_element_type=jnp.float32)
    rg_pre = g[:, HD:2 * HD] + jnp.dot(hh, whr[...],
                                       preferred_element_type=jnp.float32)
    xh_pre = g[:, 2 * HD:]
    o_dec[...] = _split2(dvv * dp)
    o_zg[...] = _split2(dvv * zg_pre)
    o_rg[...] = _split2(dvv * rg_pre)
    o_xh[...] = _split2(dvv * xh_pre)


def _s4_body(rdec, rzg, rrg, rxh, xdec, xzg, xrg, xxh, dv, bdec, bzz, brr,
             h, whh, o_zdec, o_zg, o_xhagg, o_xshh):
    dvv = lax.rsqrt(dv[...] + 1.0)
    o_zdec[...] = dvv * _cat2(rdec[...] + xdec[...]) + bdec[...]
    zg = jax.nn.sigmoid(dvv * _cat2(rzg[...] + xzg[...]) + bzz[...])
    rg = jax.nn.sigmoid(dvv * _cat2(rrg[...] + xrg[...]) + brr[...])
    o_zg[...] = zg
    o_xhagg[...] = dvv * _cat2(rxh[...] + xxh[...])
    o_xshh[...] = _split2(dvv * jnp.dot(rg * h[...], whh[...],
                                        preferred_element_type=jnp.float32))


def _s5_body(rhh, xhh, dv, xhagg, bxhh, zg, h, o_h):
    ht = jnp.tanh(xhagg[...] + lax.rsqrt(dv[...] + 1.0)
                  * (_catsum(rhh[...]) + _cat2(xhh[...])) + bxhh[...])
    zgv = zg[...]
    o_h[...] = zgv * h[...] + (1.0 - zgv) * ht


def _f32(shape):
    return jax.ShapeDtypeStruct(shape, jnp.float32)


def kernel(x, edge_all_list, edge_idx_list, edge_droped_idx_list, params):
    p = params
    ei = edge_idx_list.astype(jnp.int32)

    pad_src = jnp.zeros((EP - E,), jnp.int32)
    pad_dst = jnp.full((EP - E,), N, jnp.int32)
    srcs = [jnp.concatenate([ei[t, 0], pad_src]).reshape(NB, 128)
            for t in range(T)]
    dsts = [jnp.concatenate([ei[t, 1], pad_dst]).reshape(NB, 128)
            for t in range(T)]

    xp = jnp.pad(x, ((0, 0), (0, NP - N), (0, 0))).reshape(T * NP, XD)
    eps_p = jnp.pad(p['eps1'], ((0, NP - N), (0, 0)))

    wenc_t, wenc_b = p['Wenc'][:HD], p['Wenc'][HD:]
    wgt = jnp.concatenate([p['Wxz'][:HD], p['Wxr'][:HD], p['Wxh'][:HD]], 1)
    wxb = jnp.concatenate([p['Wxz'][HD:], p['Wxr'][HD:], p['Wxh'][HD:]], 1)
    wems = jnp.concatenate([p['Wem'], p['Wes']], 1)
    bems = jnp.concatenate([p['bem'], p['bes']])[None, :]
    bzz = (p['bxz'] + p['bhz'])[None, :]
    brr = (p['bxr'] + p['bhr'])[None, :]
    bxhh = (p['bxh'] + p['bhh'])[None, :]
    bpx = p['bpx'][None, :]
    bpz = p['bpz'][None, :]
    benc = p['benc'][None, :]
    bdec = p['bdec'][None, :]

    deg_flat = _deg_dinv(dsts[0], dsts[1], dsts[2])
    deg_b = deg_flat.reshape(T, NP, 128)

    a1_all, gxp_all = pl.pallas_call(
        _pre_body,
        grid=(T * NP // RB,),
        in_specs=[_row_spec(XD), _full_spec((XD, HD)), _full_spec((1, HD)),
                  _full_spec((HD, HD)), _full_spec((HD, 3 * HD))],
        out_specs=[_row_spec(HD), _row_spec(3 * HD)],
        out_shape=[_f32((T * NP, HD)), _f32((T * NP, 3 * HD))],
    )(xp, p['Wpx'], bpx, wenc_t, wgt)
    a1_all = a1_all.reshape(T, NP, HD)
    gxp_all = gxp_all.reshape(T, NP, 3 * HD)

    h = jnp.zeros((NP, HD), jnp.float32)
    zdecs = []
    for t in range(T):
        dv = deg_b[t]
        src, dst = srcs[t], dsts[t]

        xs_enc = pl.pallas_call(
            _s1_body,
            grid=(GRID,),
            in_specs=[_row_spec(HD), _row_spec(HD), _full_spec((HD, HD)),
                      _row_spec(128)],
            out_specs=_half_spec(),
            out_shape=_f32((2, NP, 64)),
        )(a1_all[t], h, wenc_b, dv)

        (r_enc,) = _agg1(xs_enc, src, dst)

        xs_ms = pl.pallas_call(
            _s2_body,
            grid=(GRID,),
            in_specs=[_rawh_spec(), _half_spec(), _row_spec(128),
                      _full_spec((1, HD)), _full_spec((HD, 2 * ZD))],
            out_specs=_half_spec(),
            out_shape=_f32((2, NP, 64)),
        )(r_enc, xs_enc, dv, benc, wems)

        (r_ms,) = _agg1(xs_ms, src, dst)

        xs_dec, xs_zg, xs_rg, xs_xh = pl.pallas_call(
            _s3_body,
            grid=(GRID,),
            in_specs=[_rawh_spec(), _half_spec(), _row_spec(128),
                      _full_spec((1, 2 * ZD)), _row_spec(ZD),
                      _full_spec((ZD, HD)), _full_spec((1, HD)),
                      _full_spec((ZD, HD)), _row_spec(3 * HD),
                      _full_spec((HD, 3 * HD)), _row_spec(HD),
                      _full_spec((HD, HD)), _full_spec((HD, HD))],
            out_specs=[_half_spec()] * 4,
            out_shape=[_f32((2, NP, 64))] * 4,
        )(r_ms, xs_ms, dv, bems, eps_p, p['Wpz'], bpz, p['Wdec'],
          gxp_all[t], wxb, h, p['Whz'], p['Whr'])

        r_dec, r_zg, r_rg, r_xh = _agg4(xs_dec, xs_zg, xs_rg, xs_xh, src, dst)

        zdec_t, zg, xh_agg, xs_hh = pl.pallas_call(
            _s4_body,
            grid=(GRID,),
            in_specs=[_half_spec()] * 4 + [_half_spec()] * 4
                     + [_row_spec(128), _full_spec((1, HD)),
                        _full_spec((1, HD)), _full_spec((1, HD)),
                        _row_spec(HD), _full_spec((HD, HD))],
            out_specs=[_row_spec(HD), _row_spec(HD), _row_spec(HD),
                       _half_spec()],
            out_shape=[_f32((NP, HD)), _f32((NP, HD)), _f32((NP, HD)),
                       _f32((2, NP, 64))],
        )(r_dec, r_zg, r_rg, r_xh, xs_dec, xs_zg, xs_rg, xs_xh, dv,
          bdec, bzz, brr, h, p['Whh'])

        (r_hh,) = _agg1(xs_hh, src, dst)

        h = pl.pallas_call(
            _s5_body,
            grid=(GRID,),
            in_specs=[_rawh_spec(), _half_spec(), _row_spec(128),
                      _row_spec(HD), _full_spec((1, HD)), _row_spec(HD),
                      _row_spec(HD)],
            out_specs=_row_spec(HD),
            out_shape=_f32((NP, HD)),
        )(r_hh, xs_hh, dv, xh_agg, bxhh, zg, h)

        zdecs.append(zdec_t[:N])

    return jnp.stack(zdecs)
